# baseline (speedup 1.0000x reference)
import jax
import jax.numpy as jnp
from jax import lax
from jax.experimental import pallas as pl
from jax.experimental.pallas import tpu as pltpu
from jax.experimental.pallas import tpu_sc as plsc

N_LATENT = 16
D = N_LATENT * N_LATENT
B = 16384
NC, NS, L = 2, 16, 16
NW = NC * NS
BPW = B // NW
CH = 128
NCHUNK = BPW // CH
UROW = BPW * N_LATENT // 128
OROW = CH * N_LATENT // 128

_mesh = plsc.VectorSubcoreMesh(
    core_axis_name="c", subcore_axis_name="s", num_cores=NC, num_subcores=NS)
_params = pltpu.CompilerParams(
    needs_layout_passes=False, use_tc_tiling_on_sc=True)


NH3 = 100000
GS = 16
NGRP_PAD = 12504
NSTEP = 25


def _a_body(u_hbm, idx_hbm, tab_hbm, h3_hbm, part_hbm, as_hbm, h3lin_hbm,
            idx_v, u_v, out_v, h2_v, rows_v, st0_v, st1_v, cmp0_v, cmp1_v,
            gsem, ssem, osem, t0sem, t1sem, c0sem, c1sem):
    wid = lax.axis_index("s") * NC + lax.axis_index("c")
    base = wid * BPW

    pltpu.sync_copy(idx_hbm.at[pl.ds(base, BPW)], idx_v)
    i16 = lax.iota(jnp.int32, L)

    gcp = pltpu.async_copy(tab_hbm.at[idx_v.at[pl.ds(0, CH)]],
                           rows_v.at[pl.ds(0, CH)], gsem)
    pltpu.sync_copy(u_hbm.at[pl.ds(wid * UROW, UROW)], u_v)

    def grp0_of(t):
        return jnp.minimum((t * NW + wid) * GS, 12480)

    def stage(t, st, sem):
        return pltpu.async_copy(h3_hbm.at[pl.ds(grp0_of(t) * 8, GS * 8)],
                                st, sem)

    def wait_t(st, sem):
        pltpu.make_async_copy(h3_hbm.at[pl.ds(0, GS * 8)], st, sem).wait()

    def wait_c(cmp_buf, sem):
        pltpu.make_async_copy(cmp_buf, h3lin_hbm.at[pl.ds(0, GS)],
                              sem).wait()

    def repack(st, cmp_buf, nrow=GS * 8):
        for g in range((nrow + 7) // 8):
            for j in range(8):
                r = g * 8 + j
                if r < nrow:
                    cmp_buf[g, pl.ds(j * L, L)] = st[r, pl.ds(0, L)]

    def flush(t, cmp_buf, sem):
        return pltpu.async_copy(cmp_buf, h3lin_hbm.at[pl.ds(grp0_of(t), GS)],
                                sem)

    stage(0, st0_v, t0sem)
    stage(1, st1_v, t1sem)
    wait_t(st0_v, t0sem); repack(st0_v, cmp0_v); flush(0, cmp0_v, c0sem)
    stage(2, st0_v, t0sem)
    wait_t(st1_v, t1sem); repack(st1_v, cmp1_v); flush(1, cmp1_v, c1sem)
    stage(3, st1_v, t1sem)

    def phase0(i, _):
        wait_t(st0_v, t0sem); wait_c(cmp0_v, c0sem)
        repack(st0_v, cmp0_v); flush(2 * i, cmp0_v, c0sem)
        stage(2 * i + 2, st0_v, t0sem)
        wait_t(st1_v, t1sem); wait_c(cmp1_v, c1sem)
        repack(st1_v, cmp1_v); flush(2 * i + 1, cmp1_v, c1sem)
        stage(2 * i + 3, st1_v, t1sem)
        return _

    lax.fori_loop(1, 11, phase0, 0)
    wait_t(st0_v, t0sem); wait_c(cmp0_v, c0sem)
    repack(st0_v, cmp0_v); flush(22, cmp0_v, c0sem)
    stage(24, st0_v, t0sem)
    wait_t(st1_v, t1sem); wait_c(cmp1_v, c1sem)
    repack(st1_v, cmp1_v); flush(23, cmp1_v, c1sem)
    tail_rows = NH3 - 99904
    pltpu.async_copy(h3_hbm.at[pl.ds(99904, tail_rows)],
                     st1_v.at[pl.ds(0, tail_rows)], t1sem)
    wait_t(st0_v, t0sem); wait_c(cmp0_v, c0sem)
    repack(st0_v, cmp0_v); flush(24, cmp0_v, c0sem)
    pltpu.make_async_copy(h3_hbm.at[pl.ds(0, tail_rows)],
                          st1_v.at[pl.ds(0, tail_rows)], t1sem).wait()
    wait_c(cmp1_v, c1sem)
    repack(st1_v, cmp1_v, nrow=tail_rows)
    pltpu.async_copy(cmp1_v, h3lin_hbm.at[pl.ds(12488, GS)], c1sem)
    wait_c(cmp0_v, c0sem)
    wait_c(cmp1_v, c1sem)

    scatter_cps = [None] * NCHUNK
    ocp = None
    for c in range(NCHUNK):
        cb = c % 2
        gcp.wait()
        if c >= 1:
            scatter_cps[c - 1].wait()
        if c + 1 < NCHUNK:
            nb = (c + 1) % 2
            gcp = pltpu.async_copy(
                tab_hbm.at[idx_v.at[pl.ds((c + 1) * CH, CH)]],
                rows_v.at[pl.ds(nb * CH, CH)], gsem)
        scatter_cps[c] = pltpu.async_copy(
            rows_v.at[pl.ds(cb * CH, CH)],
            as_hbm.at[pl.ds(base + c * CH, CH)], ssem)
        if ocp is not None:
            ocp.wait()

        def group(s, _, c=c, cb=cb):
            i_l = i16 + s * L
            i_r = i_l + cb * CH
            sb16 = (i_l + c * CH) * L
            acc = [None] * L
            for k in range(L):
                rv = (i16 + k) & (L - 1)
                f = sb16 + rv
                urot = plsc.load_gather(
                    u_v, [lax.shift_right_logical(f, 7), f & 127])
                for g in range(L):
                    a = plsc.load_gather(rows_v, [i_r, rv + (L * g)])
                    p = a * urot
                    acc[g] = p if acc[g] is None else acc[g] + p
            for g in range(L):
                h2_v[g, pl.ds(0, L)] = acc[g]
            for k in range(L):
                rv = (i16 + k) & (L - 1)
                f = sb16 + rv
                urot = plsc.load_gather(
                    u_v, [lax.shift_right_logical(f, 7), f & 127])
                h2rot = plsc.load_gather(h2_v, [rv, i16])
                fo = (i_l * L) + rv
                plsc.store_scatter(
                    out_v, [lax.shift_right_logical(fo, 7), fo & 127],
                    urot + h2rot)
            return _

        lax.fori_loop(0, CH // L, group, 0)
        ocp = pltpu.async_copy(
            out_v, part_hbm.at[pl.ds(wid * UROW + c * OROW, OROW)], osem)

    ocp.wait()
    scatter_cps[NCHUNK - 1].wait()


_run_a = pl.kernel(
    _a_body,
    out_type=[
        jax.ShapeDtypeStruct((B * N_LATENT // 128, 128), jnp.float32),
        jax.ShapeDtypeStruct((B, D), jnp.float32),
        jax.ShapeDtypeStruct((NGRP_PAD, 128), jnp.float32),
    ],
    mesh=_mesh,
    compiler_params=_params,
    scratch_types=[
        pltpu.VMEM((BPW,), jnp.int32),
        pltpu.VMEM((UROW, 128), jnp.float32),
        pltpu.VMEM((OROW, 128), jnp.float32),
        pltpu.VMEM((L, 128), jnp.float32),
        pltpu.VMEM((2 * CH, D), jnp.float32),
        pltpu.VMEM((GS * 8, N_LATENT), jnp.float32),
        pltpu.VMEM((GS * 8, N_LATENT), jnp.float32),
        pltpu.VMEM((GS, 128), jnp.float32),
        pltpu.VMEM((GS, 128), jnp.float32),
        pltpu.SemaphoreType.DMA,
        pltpu.SemaphoreType.DMA,
        pltpu.SemaphoreType.DMA,
        pltpu.SemaphoreType.DMA,
        pltpu.SemaphoreType.DMA,
        pltpu.SemaphoreType.DMA,
        pltpu.SemaphoreType.DMA,
    ],
)


def _h3_body(part_hbm, idx_hbm, h3_hbm, out_hbm,
             idx_v, idx8_v, part_v, h3r_v, out_v, hsem):
    wid = lax.axis_index("s") * NC + lax.axis_index("c")
    base = wid * BPW

    pltpu.sync_copy(idx_hbm.at[pl.ds(base, BPW)], idx_v)
    i16 = lax.iota(jnp.int32, L)

    def make_idx8(c):
        for s in range(CH // L):
            i_sf = i16 + (c * CH + s * L)
            v = plsc.load_gather(idx_v, [i_sf])
            plsc.store_scatter(idx8_v, [i_sf],
                               lax.shift_right_logical(v, 3))

    make_idx8(0)
    hcp = pltpu.async_copy(h3_hbm.at[idx8_v.at[pl.ds(0, CH)]],
                           h3r_v.at[pl.ds(0, CH)], hsem)
    pltpu.sync_copy(part_hbm.at[pl.ds(wid * UROW, UROW)], part_v)

    for c in range(NCHUNK):
        cb = c % 2
        if c + 1 < NCHUNK:
            make_idx8(c + 1)
        hcp.wait()
        if c + 1 < NCHUNK:
            nb = (c + 1) % 2
            hcp = pltpu.async_copy(
                h3_hbm.at[idx8_v.at[pl.ds((c + 1) * CH, CH)]],
                h3r_v.at[pl.ds(nb * CH, CH)], hsem)

        def group(s, _, c=c, cb=cb):
            i_l = i16 + s * L
            i_r = i_l + cb * CH
            i_sf = i_l + c * CH
            idxv = plsc.load_gather(idx_v, [i_sf])
            offv = (idxv & 7) * L
            sb16 = i_sf * L
            for k in range(L):
                rv = (i16 + k) & (L - 1)
                f = sb16 + rv
                prot = plsc.load_gather(
                    part_v, [lax.shift_right_logical(f, 7), f & 127])
                hrot = plsc.load_gather(h3r_v, [i_r, offv + rv])
                plsc.store_scatter(
                    out_v, [lax.shift_right_logical(f, 7), f & 127],
                    prot + hrot)
            return _

        lax.fori_loop(0, CH // L, group, 0)

    pltpu.sync_copy(out_v, out_hbm.at[pl.ds(wid * UROW, UROW)])


_run_h3 = pl.kernel(
    _h3_body,
    out_type=[
        jax.ShapeDtypeStruct((B * N_LATENT // 128, 128), jnp.float32),
    ],
    mesh=_mesh,
    compiler_params=_params,
    scratch_types=[
        pltpu.VMEM((BPW,), jnp.int32),
        pltpu.VMEM((BPW,), jnp.int32),
        pltpu.VMEM((UROW, 128), jnp.float32),
        pltpu.VMEM((2 * CH, 128), jnp.float32),
        pltpu.VMEM((UROW, 128), jnp.float32),
        pltpu.SemaphoreType.DMA,
    ],
)


def kernel(u, sample_covariate, As_rng, A_s_enc, h3_embed):
    idx = sample_covariate.astype(jnp.int32)
    u2 = u.reshape(B * N_LATENT // 128, 128)
    part, as2, h3lin = _run_a(u2, idx, A_s_enc, h3_embed)
    (out2,) = _run_h3(part, idx, h3lin)
    out = out2.reshape(B, N_LATENT)
    a_s = as2.reshape(B, N_LATENT, N_LATENT)
    return (out, a_s)


# speedup vs baseline: 1.3833x; 1.3833x over previous
import jax
import jax.numpy as jnp
from jax import lax
from jax.experimental import pallas as pl
from jax.experimental.pallas import tpu as pltpu
from jax.experimental.pallas import tpu_sc as plsc

N_LATENT = 16
D = N_LATENT * N_LATENT
B = 16384
NC, NS, L = 2, 16, 16
NW = NC * NS
BPW = B // NW
CH = 128
NCHUNK = BPW // CH
UROW = BPW * N_LATENT // 128
OROW = CH * N_LATENT // 128

_mesh = plsc.VectorSubcoreMesh(
    core_axis_name="c", subcore_axis_name="s", num_cores=NC, num_subcores=NS)
_params = pltpu.CompilerParams(
    needs_layout_passes=False, use_tc_tiling_on_sc=True)


def _a_body(u_hbm, idx_hbm, tab_hbm, part_hbm, as_hbm,
            idx_v, u_v, ust_v, out_v, h2_v, rows_v, gsem, ssem, osem, usem):
    wid = lax.axis_index("s") * NC + lax.axis_index("c")
    base = wid * BPW

    pltpu.sync_copy(idx_hbm.at[pl.ds(base, BPW)], idx_v)
    i16 = lax.iota(jnp.int32, L)

    gcp = pltpu.async_copy(tab_hbm.at[idx_v.at[pl.ds(0, CH)]],
                           rows_v.at[pl.ds(0, CH)], gsem)

    ucp = pltpu.async_copy(u_hbm.at[pl.ds(base, CH)],
                           ust_v.at[pl.ds(0, CH)], usem)
    for c in range(NCHUNK):
        ucp.wait()
        if c + 1 < NCHUNK:
            nb = (c + 1) % 2
            ucp = pltpu.async_copy(u_hbm.at[pl.ds(base + (c + 1) * CH, CH)],
                                   ust_v.at[pl.ds(nb * CH, CH)], usem)
        cb = (c % 2) * CH
        for r in range(CH):
            fr = c * CH + r
            u_v[fr * L // 128, pl.ds((fr * L) % 128, L)] = \
                ust_v[cb + r, pl.ds(0, L)]

    scatter_cps = [None] * NCHUNK
    ocp = None
    for c in range(NCHUNK):
        cb = c % 2
        gcp.wait()
        if c >= 1:
            scatter_cps[c - 1].wait()
        if c + 1 < NCHUNK:
            nb = (c + 1) % 2
            gcp = pltpu.async_copy(
                tab_hbm.at[idx_v.at[pl.ds((c + 1) * CH, CH)]],
                rows_v.at[pl.ds(nb * CH, CH)], gsem)
        scatter_cps[c] = pltpu.async_copy(
            rows_v.at[pl.ds(cb * CH, CH)],
            as_hbm.at[pl.ds(base + c * CH, CH)], ssem)
        if ocp is not None:
            ocp.wait()

        def group(s, _, c=c, cb=cb):
            i_l = i16 + s * L
            i_r = i_l + cb * CH
            sb16 = (i_l + c * CH) * L
            acc = [None] * L
            for k in range(L):
                rv = (i16 + k) & (L - 1)
                f = sb16 + rv
                urot = plsc.load_gather(
                    u_v, [lax.shift_right_logical(f, 7), f & 127])
                for g in range(L):
                    a = plsc.load_gather(rows_v, [i_r, rv + (L * g)])
                    p = a * urot
                    acc[g] = p if acc[g] is None else acc[g] + p
            for g in range(L):
                h2_v[g, pl.ds(0, L)] = acc[g]
            for k in range(L):
                rv = (i16 + k) & (L - 1)
                f = sb16 + rv
                urot = plsc.load_gather(
                    u_v, [lax.shift_right_logical(f, 7), f & 127])
                h2rot = plsc.load_gather(h2_v, [rv, i16])
                fo = (i_l * L) + rv
                plsc.store_scatter(
                    out_v, [lax.shift_right_logical(fo, 7), fo & 127],
                    urot + h2rot)
            return _

        lax.fori_loop(0, CH // L, group, 0)
        ocp = pltpu.async_copy(
            out_v, part_hbm.at[pl.ds(wid * UROW + c * OROW, OROW)], osem)

    ocp.wait()
    scatter_cps[NCHUNK - 1].wait()


_run_a = pl.kernel(
    _a_body,
    out_type=[
        jax.ShapeDtypeStruct((B * N_LATENT // 128, 128), jnp.float32),
        jax.ShapeDtypeStruct((B, D), jnp.float32),
    ],
    mesh=_mesh,
    compiler_params=_params,
    scratch_types=[
        pltpu.VMEM((BPW,), jnp.int32),
        pltpu.VMEM((UROW, 128), jnp.float32),
        pltpu.VMEM((2 * CH, N_LATENT), jnp.float32),
        pltpu.VMEM((OROW, 128), jnp.float32),
        pltpu.VMEM((L, 128), jnp.float32),
        pltpu.VMEM((2 * CH, D), jnp.float32),
        pltpu.SemaphoreType.DMA,
        pltpu.SemaphoreType.DMA,
        pltpu.SemaphoreType.DMA,
        pltpu.SemaphoreType.DMA,
    ],
)


def _h3_body(part_hbm, idx_hbm, h3_hbm, out_hbm,
             idx_v, idx8_v, part_v, h3r_v, out_v, hsem):
    wid = lax.axis_index("s") * NC + lax.axis_index("c")
    base = wid * BPW

    pltpu.sync_copy(idx_hbm.at[pl.ds(base, BPW)], idx_v)
    i16 = lax.iota(jnp.int32, L)

    def make_idx8(c):
        for s in range(CH // L):
            i_sf = i16 + (c * CH + s * L)
            v = plsc.load_gather(idx_v, [i_sf])
            plsc.store_scatter(idx8_v, [i_sf],
                               lax.shift_right_logical(v, 3))

    make_idx8(0)
    hcp = pltpu.async_copy(h3_hbm.at[idx8_v.at[pl.ds(0, CH)]],
                           h3r_v.at[pl.ds(0, CH)], hsem)
    pltpu.sync_copy(part_hbm.at[pl.ds(wid * UROW, UROW)], part_v)

    for c in range(NCHUNK):
        cb = c % 2
        if c + 1 < NCHUNK:
            make_idx8(c + 1)
        hcp.wait()
        if c + 1 < NCHUNK:
            nb = (c + 1) % 2
            hcp = pltpu.async_copy(
                h3_hbm.at[idx8_v.at[pl.ds((c + 1) * CH, CH)]],
                h3r_v.at[pl.ds(nb * CH, CH)], hsem)

        def group(s, _, c=c, cb=cb):
            i_l = i16 + s * L
            i_r = i_l + cb * CH
            i_sf = i_l + c * CH
            idxv = plsc.load_gather(idx_v, [i_sf])
            offv = (idxv & 7) * L
            sb16 = i_sf * L
            for k in range(L):
                rv = (i16 + k) & (L - 1)
                f = sb16 + rv
                prot = plsc.load_gather(
                    part_v, [lax.shift_right_logical(f, 7), f & 127])
                hrot = plsc.load_gather(h3r_v, [i_r, offv + rv])
                plsc.store_scatter(
                    out_v, [lax.shift_right_logical(f, 7), f & 127],
                    prot + hrot)
            return _

        lax.fori_loop(0, CH // L, group, 0)

    pltpu.sync_copy(out_v, out_hbm.at[pl.ds(wid * UROW, UROW)])


_run_h3 = pl.kernel(
    _h3_body,
    out_type=[
        jax.ShapeDtypeStruct((B * N_LATENT // 128, 128), jnp.float32),
    ],
    mesh=_mesh,
    compiler_params=_params,
    scratch_types=[
        pltpu.VMEM((BPW,), jnp.int32),
        pltpu.VMEM((BPW,), jnp.int32),
        pltpu.VMEM((UROW, 128), jnp.float32),
        pltpu.VMEM((2 * CH, 128), jnp.float32),
        pltpu.VMEM((UROW, 128), jnp.float32),
        pltpu.SemaphoreType.DMA,
    ],
)


def kernel(u, sample_covariate, As_rng, A_s_enc, h3_embed):
    idx = sample_covariate.astype(jnp.int32)
    h3g = h3_embed.reshape(h3_embed.shape[0] // 8, 128)
    part, as2 = _run_a(u, idx, A_s_enc)
    (out2,) = _run_h3(part, idx, h3g)
    out = out2.reshape(B, N_LATENT)
    a_s = as2.reshape(B, N_LATENT, N_LATENT)
    return (out, a_s)
